# Initial kernel scaffold; baseline (speedup 1.0000x reference)
import jax
import jax.numpy as jnp
from jax.experimental import pallas as pl


def kernel(sentences_hidden, sentences_num, sentences_mask, sent_adjacent_matrix, head_type, edge_type, node_query, W_hp, b_hp, W_ql, b_ql, W_kl, b_kl, g_q, beta_q, g_k, beta_k, flag_embed, edge_embed, Wq, Wk, Wv, We):
    raise NotImplementedError("write your pallas kernel here")


# speedup vs baseline: 2.9673x; 2.9673x over previous
import functools
import math

import jax
import jax.numpy as jnp
from jax import lax
from jax.experimental import pallas as pl
from jax.experimental.pallas import tpu as pltpu

D = 512


def _pool_body(s_ref, m_ref, out_ref):
    s = s_ref[...]
    m = m_ref[...]
    ps = jnp.sum(s * m[:, :, None], axis=1)
    sl = jnp.sum(m, axis=1, keepdims=True)
    sl = jnp.where(sl != 0.0, sl, 1.0)
    out_ref[...] = ps / sl


def _pool(sentences_hidden, sentences_mask, rows_per_block=16):
    BS, L, DH = sentences_hidden.shape
    nblk = BS // rows_per_block
    return pl.pallas_call(
        _pool_body,
        grid=(nblk,),
        in_specs=[
            pl.BlockSpec((rows_per_block, L, DH), lambda i: (i, 0, 0)),
            pl.BlockSpec((rows_per_block, L), lambda i: (i, 0)),
        ],
        out_specs=pl.BlockSpec((rows_per_block, DH), lambda i: (i, 0)),
        out_shape=jax.ShapeDtypeStruct((BS, DH), jnp.float32),
    )(sentences_hidden, sentences_mask)


def _dense_body(pooled_ref, ht_ref, nq_ref, W_hp_ref, b_hp_ref, W_ql_ref,
                b_ql_ref, g_q_ref, beta_q_ref, flag_ref, edge_ref,
                Wq_ref, Wk_ref, Wv_ref, We_ref,
                h_ref, q_ref, k_ref, v_ref, ew_ref, query_ref):
    pooled = pooled_ref[...]
    node = jnp.dot(pooled, W_hp_ref[...],
                   preferred_element_type=jnp.float32) + b_hp_ref[...]
    ht = ht_ref[...].astype(jnp.float32)
    f0 = flag_ref[0:1, :]
    f1 = flag_ref[1:2, :]
    h = node + f0 + ht * (f1 - f0)
    h_ref[...] = h
    q_ref[...] = jnp.dot(h, Wq_ref[...], preferred_element_type=jnp.float32)
    k_ref[...] = jnp.dot(h, Wk_ref[...], preferred_element_type=jnp.float32)
    v_ref[...] = jnp.dot(h, Wv_ref[...], preferred_element_type=jnp.float32)
    ew_ref[...] = jnp.dot(edge_ref[...], We_ref[...],
                          preferred_element_type=jnp.float32)
    ql = jnp.dot(nq_ref[...], W_ql_ref[...],
                 preferred_element_type=jnp.float32) + b_ql_ref[...]
    mu = jnp.mean(ql, axis=-1, keepdims=True)
    var = jnp.mean((ql - mu) ** 2, axis=-1, keepdims=True)
    query_ref[...] = ((ql - mu) / jnp.sqrt(var + 1e-5)) * g_q_ref[...] \
        + beta_q_ref[...]


def _dense(pooled, head_flat, node_query, W_hp, b_hp, W_ql, b_ql, g_q,
           beta_q, flag_embed, edge_embed, Wq, Wk, Wv, We):
    BS, DH = pooled.shape
    B = node_query.shape[0]
    outs = (
        jax.ShapeDtypeStruct((BS, D), jnp.float32),
        jax.ShapeDtypeStruct((BS, D), jnp.float32),
        jax.ShapeDtypeStruct((BS, D), jnp.float32),
        jax.ShapeDtypeStruct((BS, D), jnp.float32),
        jax.ShapeDtypeStruct((5, D), jnp.float32),
        jax.ShapeDtypeStruct((B, D), jnp.float32),
    )
    return pl.pallas_call(_dense_body, out_shape=outs)(
        pooled, head_flat, node_query, W_hp, b_hp, W_ql, b_ql, g_q, beta_q,
        flag_embed, edge_embed, Wq, Wk, Wv, We)


def _attn_body(h_ref, q_ref, k_ref, v_ref, adj_ref, et_ref, ew_ref,
               query_ref, mask_ref, W_kl_ref, b_kl_ref, g_k_ref,
               beta_k_ref, hidden_ref, recall_ref, *, S):
    h = h_ref[0]
    q = q_ref[0]
    k = k_ref[0]
    v = v_ref[0]
    adj = adj_ref[0]
    et = et_ref[0]
    dn = (((1,), (1,)), ((), ()))
    scores = lax.dot_general(q, k, dn,
                             preferred_element_type=jnp.float32)
    qe = lax.dot_general(q, ew_ref[...], dn,
                         preferred_element_type=jnp.float32)
    esc = jnp.zeros_like(scores)
    for t in range(5):
        esc = jnp.where(et == t, jnp.broadcast_to(qe[:, t:t + 1],
                                                  scores.shape), esc)
    scores = (scores + esc) * (1.0 / math.sqrt(float(D)))
    neg = jnp.float32(-1e9)
    scores = jnp.where(adj > 0, scores, neg)
    mx = jnp.max(scores, axis=-1, keepdims=True)
    p = jnp.exp(scores - mx)
    attn = p / jnp.sum(p, axis=-1, keepdims=True)
    row_has = (jnp.sum(adj.astype(jnp.float32), axis=-1, keepdims=True)
               > 0.0).astype(jnp.float32)
    attn = attn * row_has
    hidden = jnp.dot(attn, v, preferred_element_type=jnp.float32) + h
    hidden_ref[0] = hidden
    kl = jnp.dot(hidden, W_kl_ref[...],
                 preferred_element_type=jnp.float32) + b_kl_ref[...]
    mu = jnp.mean(kl, axis=-1, keepdims=True)
    var = jnp.mean((kl - mu) ** 2, axis=-1, keepdims=True)
    key = ((kl - mu) / jnp.sqrt(var + 1e-5)) * g_k_ref[...] + beta_k_ref[...]
    logits = jnp.sum(key * query_ref[0], axis=-1)
    pad = (jnp.sum(mask_ref[0], axis=-1) != 0.0).astype(jnp.float32)
    recall_ref[0] = (jax.nn.sigmoid(logits) * pad)[None, :]


def _attn(h, q, k, v, adj, et, ew, query3, mask3, W_kl, b_kl, g_k, beta_k):
    B, S, _ = h.shape
    L = mask3.shape[-1]
    bsd = pl.BlockSpec((1, S, D), lambda b: (b, 0, 0))
    bss = pl.BlockSpec((1, S, S), lambda b: (b, 0, 0))
    full = lambda shape: pl.BlockSpec(shape, lambda b: tuple(0 for _ in shape))
    outs = (
        jax.ShapeDtypeStruct((B, S, D), jnp.float32),
        jax.ShapeDtypeStruct((B, 1, S), jnp.float32),
    )
    return pl.pallas_call(
        functools.partial(_attn_body, S=S),
        grid=(B,),
        in_specs=[bsd, bsd, bsd, bsd, bss, bss,
                  full((5, D)),
                  pl.BlockSpec((1, 1, D), lambda b: (b, 0, 0)),
                  pl.BlockSpec((1, S, L), lambda b: (b, 0, 0)),
                  full((D, D)), full((1, D)), full((1, D)), full((1, D))],
        out_specs=[bsd, pl.BlockSpec((1, 1, S), lambda b: (b, 0, 0))],
        out_shape=outs,
    )(h, q, k, v, adj, et, ew, query3, mask3, W_kl, b_kl, g_k, beta_k)


def kernel(sentences_hidden, sentences_num, sentences_mask,
           sent_adjacent_matrix, head_type, edge_type, node_query,
           W_hp, b_hp, W_ql, b_ql, W_kl, b_kl, g_q, beta_q, g_k, beta_k,
           flag_embed, edge_embed, Wq, Wk, Wv, We):
    BS, L, DH = sentences_hidden.shape
    B = sentences_num.shape[0]
    S = BS // B

    pooled = _pool(sentences_hidden, sentences_mask)

    head_flat = head_type.reshape(BS, 1).astype(jnp.int32)
    r1 = lambda x: x.reshape(1, -1)
    h, q, k, v, ew, query = _dense(
        pooled, head_flat, node_query, W_hp, r1(b_hp), W_ql, r1(b_ql),
        r1(g_q), r1(beta_q), flag_embed, edge_embed, Wq, Wk, Wv, We)

    h3 = h.reshape(B, S, D)
    q3 = q.reshape(B, S, D)
    k3 = k.reshape(B, S, D)
    v3 = v.reshape(B, S, D)
    adj = sent_adjacent_matrix.astype(jnp.int32)
    et = edge_type.astype(jnp.int32)
    mask3 = sentences_mask.reshape(B, S, L)
    hidden, recall3 = _attn(h3, q3, k3, v3, adj, et, ew,
                            query.reshape(B, 1, D), mask3,
                            W_kl, r1(b_kl), r1(g_k), r1(beta_k))
    return recall3.reshape(B, S), hidden
